# baseline (speedup 1.0000x reference)
import jax
import jax.numpy as jnp
from jax import lax
from jax.experimental import pallas as pl
from jax.experimental.pallas import tpu as pltpu
from jax.experimental.pallas import tpu_sc as plsc

_NC = 2
_NS = 16
_NW = _NC * _NS
_SCUT = 3072
_BS = 1024


def _sc_body(x_hbm, t_hbm, o_hbm, t_buf, io0, io1, si0, si1, so0, so1):
    S, D = t_hbm.shape
    B = x_hbm.shape[0] // S
    wrows = (S - _SCUT) // _NW
    wid = lax.axis_index("s") * _NC + lax.axis_index("c")
    s0 = _SCUT + wid * wrows
    ios = (io0, io1)
    sin = (si0, si1)
    sout = (so0, so1)

    def issue_in(b):
        return pltpu.async_copy(
            x_hbm.at[pl.ds(b * S + s0, wrows)], ios[b % 2], sin[b % 2]
        )

    def issue_out(b):
        return pltpu.async_copy(
            ios[b % 2], o_hbm.at[pl.ds(b * S + s0, wrows)], sout[b % 2]
        )

    pending_in = {b: issue_in(b) for b in range(min(2, B))}
    pltpu.sync_copy(t_hbm.at[pl.ds(s0, wrows)], t_buf)
    pending_out = {}
    for b in range(B):
        pending_in.pop(b).wait()
        io = ios[b % 2]
        for r in range(wrows):

            @plsc.parallel_loop(0, D, step=16, unroll=8)
            def _add(k):
                io[r, pl.ds(k, 16)] = io[r, pl.ds(k, 16)] + t_buf[r, pl.ds(k, 16)]

        pending_out[b] = issue_out(b)
        if b + 2 < B:
            pending_out.pop(b).wait()
            pending_in[b + 2] = issue_in(b + 2)
    for b in sorted(pending_out):
        pending_out.pop(b).wait()


def _tc_body(alias_ref, x_ref, t_ref, o_ref):
    del alias_ref
    o_ref[...] = x_ref[...] + t_ref[...]


def kernel(inputs, pos_embedding_table):
    B, S, D = inputs.shape
    x = inputs.reshape(B * S, D)
    mesh = plsc.VectorSubcoreMesh(core_axis_name="c", subcore_axis_name="s")
    wrows = (S - _SCUT) // _NW
    sc_out = pl.kernel(
        _sc_body,
        out_type=jax.ShapeDtypeStruct((B * S, D), inputs.dtype),
        mesh=mesh,
        scratch_types=(
            [pltpu.VMEM((wrows, D), jnp.float32)] * 3
            + [pltpu.SemaphoreType.DMA] * 4
        ),
    )(x, pos_embedding_table)

    nsb = S // _BS
    out = pl.pallas_call(
        _tc_body,
        grid=(_SCUT // _BS, B),
        in_specs=[
            pl.BlockSpec(memory_space=pl.ANY),
            pl.BlockSpec((_BS, D), lambda i, b: (b * nsb + i, 0)),
            pl.BlockSpec((_BS, D), lambda i, b: (i, 0)),
        ],
        out_specs=pl.BlockSpec((_BS, D), lambda i, b: (b * nsb + i, 0)),
        out_shape=jax.ShapeDtypeStruct((B * S, D), inputs.dtype),
        input_output_aliases={0: 0},
    )(sc_out, x, pos_embedding_table)
    return out.reshape(B, S, D)


# speedup vs baseline: 1.0366x; 1.0366x over previous
import jax
import jax.numpy as jnp
from jax import lax
from jax.experimental import pallas as pl
from jax.experimental.pallas import tpu as pltpu
from jax.experimental.pallas import tpu_sc as plsc

_NC = 2
_NS = 16
_NW = _NC * _NS
_SCUT = 3840
_BS = 384


def _sc_body(x_hbm, t_hbm, o_hbm, t_buf, io0, io1, si0, si1, so0, so1):
    S, D = t_hbm.shape
    B = x_hbm.shape[0] // S
    wrows = (S - _SCUT) // _NW
    wid = lax.axis_index("s") * _NC + lax.axis_index("c")
    s0 = _SCUT + wid * wrows
    ios = (io0, io1)
    sin = (si0, si1)
    sout = (so0, so1)

    def issue_in(b):
        return pltpu.async_copy(
            x_hbm.at[pl.ds(b * S + s0, wrows)], ios[b % 2], sin[b % 2]
        )

    def issue_out(b):
        return pltpu.async_copy(
            ios[b % 2], o_hbm.at[pl.ds(b * S + s0, wrows)], sout[b % 2]
        )

    pending_in = {b: issue_in(b) for b in range(min(2, B))}
    pltpu.sync_copy(t_hbm.at[pl.ds(s0, wrows)], t_buf)
    pending_out = {}
    for b in range(B):
        pending_in.pop(b).wait()
        io = ios[b % 2]
        for r in range(wrows):

            @plsc.parallel_loop(0, D, step=16, unroll=8)
            def _add(k):
                io[r, pl.ds(k, 16)] = io[r, pl.ds(k, 16)] + t_buf[r, pl.ds(k, 16)]

        pending_out[b] = issue_out(b)
        if b + 2 < B:
            pending_out.pop(b).wait()
            pending_in[b + 2] = issue_in(b + 2)
    for b in sorted(pending_out):
        pending_out.pop(b).wait()


def _tc_body(alias_ref, x_ref, t_ref, o_ref):
    del alias_ref
    o_ref[...] = x_ref[...] + t_ref[...]


def kernel(inputs, pos_embedding_table):
    B, S, D = inputs.shape
    x = inputs.reshape(B * S, D)
    mesh = plsc.VectorSubcoreMesh(core_axis_name="c", subcore_axis_name="s")
    wrows = (S - _SCUT) // _NW
    sc_out = pl.kernel(
        _sc_body,
        out_type=jax.ShapeDtypeStruct((B * S, D), inputs.dtype),
        mesh=mesh,
        scratch_types=(
            [pltpu.VMEM((wrows, D), jnp.float32)] * 3
            + [pltpu.SemaphoreType.DMA] * 4
        ),
    )(x, pos_embedding_table)

    nsb = S // _BS
    out = pl.pallas_call(
        _tc_body,
        grid=(_SCUT // _BS, B),
        in_specs=[
            pl.BlockSpec(memory_space=pl.ANY),
            pl.BlockSpec((_BS, D), lambda i, b: (b * nsb + i, 0)),
            pl.BlockSpec((_BS, D), lambda i, b: (i, 0)),
        ],
        out_specs=pl.BlockSpec((_BS, D), lambda i, b: (b * nsb + i, 0)),
        out_shape=jax.ShapeDtypeStruct((B * S, D), inputs.dtype),
        input_output_aliases={0: 0},
    )(sc_out, x, pos_embedding_table)
    return out.reshape(B, S, D)
